# Initial kernel scaffold; baseline (speedup 1.0000x reference)
import jax
import jax.numpy as jnp
from jax.experimental import pallas as pl


def kernel(logits_target, labels_target):
    raise NotImplementedError("write your pallas kernel here")


# speedup vs baseline: 9.9386x; 9.9386x over previous
import jax
import jax.numpy as jnp
from jax import lax
from jax.experimental import pallas as pl
from jax.experimental.pallas import tpu as pltpu
from jax.experimental.pallas import tpu_sc as plsc

N = 65536
C = 100
NBINS = 15
SHIFT = 16
NBUCKETS = 16384

NBA = 1024


def _softmax_body(logits_ref, conft_ref, idxt_ref, minv_ref, maxv_ref):
    i = pl.program_id(0)
    x = logits_ref[...]
    m = jnp.max(x, axis=1, keepdims=True)
    e = jnp.exp(x - m)
    s = jnp.sum(e, axis=1, keepdims=True)
    conf = e / s
    confp = jnp.pad(conf, ((0, 0), (0, 128 - C)))
    conft = confp.T[0:C, :]
    conft_ref[...] = conft
    bits = lax.bitcast_convert_type(conft, jnp.int32)
    bucket = lax.shift_right_logical(bits, SHIFT)
    rows = lax.broadcasted_iota(jnp.int32, (C, 1), 0)
    local = jnp.where(rows >= C // 2, rows - C // 2, rows)
    idxt_ref[...] = bucket + local * NBUCKETS
    mn = jnp.min(conft, axis=1, keepdims=True)
    mx = jnp.max(conft, axis=1, keepdims=True)

    @pl.when(i == 0)
    def _():
        minv_ref[...] = mn
        maxv_ref[...] = mx

    @pl.when(i > 0)
    def _():
        minv_ref[...] = jnp.minimum(minv_ref[...], mn)
        maxv_ref[...] = jnp.maximum(maxv_ref[...], mx)


def _softmax_call(logits):
    return pl.pallas_call(
        _softmax_body,
        grid=(N // NBA,),
        in_specs=[pl.BlockSpec((NBA, C), lambda i: (i, 0))],
        out_specs=[
            pl.BlockSpec((C, NBA), lambda i: (0, i)),
            pl.BlockSpec((C, NBA), lambda i: (0, i)),
            pl.BlockSpec((C, 1), lambda i: (0, 0)),
            pl.BlockSpec((C, 1), lambda i: (0, 0)),
        ],
        out_shape=[
            jax.ShapeDtypeStruct((C, N), jnp.float32),
            jax.ShapeDtypeStruct((C, N), jnp.int32),
            jax.ShapeDtypeStruct((C, 1), jnp.float32),
            jax.ShapeDtypeStruct((C, 1), jnp.float32),
        ],
    )(logits)


NCORES = 2
NSUB = 16
CLS_PER_CORE = C // NCORES
CH = 8192
NCHUNK = N // CH
ROWS = CH // 128


def _hist_body(idx3_hbm, ones_hbm, zeros_hbm, out_hbm,
               idx_v, stage_v, ones_v, sem, hist_sh):
    cid = lax.axis_index("c")
    sid = lax.axis_index("s")
    cnt = jnp.where(sid < 2, 4, 3)
    start_local = 3 * sid + jnp.minimum(sid, 2)
    core_base = cid * CLS_PER_CORE

    pltpu.sync_copy(ones_hbm, ones_v)

    def do_class(z):
        local = start_local + z
        cls = core_base + local
        hbase = local * NBUCKETS
        pltpu.sync_copy(zeros_hbm, hist_sh.at[pl.ds(hbase, NBUCKETS)])
        for ch in range(NCHUNK):
            pltpu.sync_copy(idx3_hbm.at[cls, pl.ds(ch * ROWS, ROWS)], idx_v)
            for g in range(ROWS // 16):
                descs = [
                    pltpu.async_copy(
                        ones_v.at[g * 16 + r],
                        hist_sh.at[plsc.Indices(idx_v.at[g * 16 + r])],
                        sem, add=True)
                    for r in range(16)
                ]
                for d in descs:
                    d.wait()
        pltpu.sync_copy(hist_sh.at[pl.ds(hbase, NBUCKETS)], stage_v)
        pltpu.sync_copy(stage_v, out_hbm.at[cls])

    def loop_body(z, _):
        @pl.when(z < cnt)
        def _():
            do_class(z)
        return 0

    lax.fori_loop(0, 4, loop_body, 0)


def _hist_call(idxt):
    mesh = plsc.VectorSubcoreMesh(core_axis_name="c", subcore_axis_name="s")
    ones = jnp.ones((ROWS, 128), jnp.float32)
    zeros = jnp.zeros((NBUCKETS,), jnp.float32)
    idx3 = idxt.reshape(C, N // 128, 128)
    fn = pl.kernel(
        _hist_body,
        out_type=jax.ShapeDtypeStruct((C, NBUCKETS), jnp.float32),
        mesh=mesh,
        scratch_types=[
            pltpu.VMEM((ROWS, 128), jnp.int32),
            pltpu.VMEM((NBUCKETS,), jnp.float32),
            pltpu.VMEM((ROWS, 128), jnp.float32),
            pltpu.SemaphoreType.DMA,
            pltpu.VMEM_SHARED((CLS_PER_CORE * NBUCKETS,), jnp.float32),
        ],
    )
    return fn(idx3, ones, zeros)


BCH = 512
NBSTEP = NBUCKETS // BCH
BIG = 3.0e38


def _edges_body(hist_ref, minv_ref, maxv_ref, edges_ref,
                carry_ref, bacc_ref, cumlo_ref, cumhi_ref):
    j = pl.program_id(0)

    @pl.when(j == 0)
    def _():
        carry_ref[...] = jnp.zeros((C, 1), jnp.float32)
        bacc_ref[...] = jnp.zeros((C, 16), jnp.float32)
        cumlo_ref[...] = jnp.full((C, 16), -1.0, jnp.float32)
        cumhi_ref[...] = jnp.full((C, 16), BIG, jnp.float32)

    h = hist_ref[...]
    rr = lax.broadcasted_iota(jnp.int32, (BCH, BCH), 0)
    cc = lax.broadcasted_iota(jnp.int32, (BCH, BCH), 1)
    tri = (rr <= cc).astype(jnp.float32)
    cum_in = jax.lax.dot(h, tri, preferred_element_type=jnp.float32)
    cum = cum_in + carry_ref[...]
    carry_ref[...] = cum[:, BCH - 1:BCH]

    zero1 = jnp.zeros((C, 1), jnp.float32)
    neg1 = jnp.full((C, 1), -1.0, jnp.float32)
    big1 = jnp.full((C, 1), BIG, jnp.float32)
    nlt_cols, mlo_cols, mhi_cols = [zero1], [neg1], [big1]
    for k in range(1, 15):
        t = k * (float(N) / NBINS) + 0.5
        lt = cum < t
        nlt_cols.append(jnp.sum(jnp.where(lt, 1.0, 0.0), axis=1,
                                keepdims=True))
        mlo_cols.append(jnp.max(jnp.where(lt, cum, -1.0), axis=1,
                                keepdims=True))
        mhi_cols.append(jnp.min(jnp.where(lt, BIG, cum), axis=1,
                                keepdims=True))
    nlt_cols.append(zero1)
    mlo_cols.append(neg1)
    mhi_cols.append(big1)
    bacc_ref[...] = bacc_ref[...] + jnp.concatenate(nlt_cols, axis=1)
    cumlo_ref[...] = jnp.maximum(cumlo_ref[...],
                                 jnp.concatenate(mlo_cols, axis=1))
    cumhi_ref[...] = jnp.minimum(cumhi_ref[...],
                                 jnp.concatenate(mhi_cols, axis=1))

    @pl.when(j == NBSTEP - 1)
    def _():
        b_i = bacc_ref[...].astype(jnp.int32)
        vlo = lax.bitcast_convert_type(b_i << SHIFT, jnp.float32)
        vhi = lax.bitcast_convert_type((b_i + 1) << SHIFT, jnp.float32)
        lo = jnp.maximum(cumlo_ref[...], 0.0)
        hi = cumhi_ref[...]
        kk = lax.broadcasted_iota(jnp.int32, (C, 16), 1).astype(jnp.float32)
        pos = kk * (float(N) / NBINS) + 0.5
        frac = (pos - lo) / jnp.maximum(hi - lo, 1.0)
        edge = vlo + frac * (vhi - vlo)
        ki = lax.broadcasted_iota(jnp.int32, (C, 16), 1)
        mn = jnp.broadcast_to(minv_ref[...], (C, 16))
        mx = jnp.broadcast_to(maxv_ref[...], (C, 16))
        edges_ref[...] = jnp.where(ki == 0, mn,
                                   jnp.where(ki == 15, mx, edge))


def _edges_call(hist, minv, maxv):
    return pl.pallas_call(
        _edges_body,
        grid=(NBSTEP,),
        in_specs=[
            pl.BlockSpec((C, BCH), lambda j: (0, j)),
            pl.BlockSpec((C, 1), lambda j: (0, 0)),
            pl.BlockSpec((C, 1), lambda j: (0, 0)),
        ],
        out_specs=pl.BlockSpec((C, 16), lambda j: (0, 0)),
        out_shape=jax.ShapeDtypeStruct((C, 16), jnp.float32),
        scratch_shapes=[
            pltpu.VMEM((C, 1), jnp.float32),
            pltpu.VMEM((C, 16), jnp.float32),
            pltpu.VMEM((C, 16), jnp.float32),
            pltpu.VMEM((C, 16), jnp.float32),
        ],
    )(hist, minv, maxv)


ND = 2048
NDSTEP = N // ND


def _bin_body(conft_ref, labels_ref, edges_ref, out_ref,
              u1_ref, uc_ref, uc2_ref, uy_ref, ucy_ref):
    i = pl.program_id(0)

    @pl.when(i == 0)
    def _():
        z = jnp.zeros((C, 16), jnp.float32)
        u1_ref[...] = z
        uc_ref[...] = z
        uc2_ref[...] = z
        uy_ref[...] = z
        ucy_ref[...] = z

    conf = conft_ref[...]
    labels = labels_ref[0]
    clsid = lax.broadcasted_iota(jnp.int32, (C, ND), 0)
    y = (labels == clsid).astype(jnp.float32)
    c2 = conf * conf
    cy = conf * y
    edges = edges_ref[...]

    cols = []
    for k in range(16):
        m = conf > edges[:, k:k + 1]
        s1 = jnp.sum(jnp.where(m, 1.0, 0.0), axis=1, keepdims=True)
        sc = jnp.sum(jnp.where(m, conf, 0.0), axis=1, keepdims=True)
        sc2 = jnp.sum(jnp.where(m, c2, 0.0), axis=1, keepdims=True)
        sy = jnp.sum(jnp.where(m, y, 0.0), axis=1, keepdims=True)
        scy = jnp.sum(jnp.where(m, cy, 0.0), axis=1, keepdims=True)
        cols.append((s1, sc, sc2, sy, scy))
    u1_ref[...] = u1_ref[...] + jnp.concatenate([c[0] for c in cols], axis=1)
    uc_ref[...] = uc_ref[...] + jnp.concatenate([c[1] for c in cols], axis=1)
    uc2_ref[...] = uc2_ref[...] + jnp.concatenate([c[2] for c in cols],
                                                  axis=1)
    uy_ref[...] = uy_ref[...] + jnp.concatenate([c[3] for c in cols], axis=1)
    ucy_ref[...] = ucy_ref[...] + jnp.concatenate([c[4] for c in cols],
                                                  axis=1)

    @pl.when(i == NDSTEP - 1)
    def _():
        u1 = u1_ref[...]
        uc = uc_ref[...]
        uc2 = uc2_ref[...]
        uy = uy_ref[...]
        ucy = ucy_ref[...]
        n_b = u1[:, 0:15] - u1[:, 1:16]
        s_c = uc[:, 0:15] - uc[:, 1:16]
        s_c2 = uc2[:, 0:15] - uc2[:, 1:16]
        s_y = uy[:, 0:15] - uy[:, 1:16]
        s_cy = ucy[:, 0:15] - ucy[:, 1:16]
        e = 1.0 / (n_b - 1.0)
        a = s_y * e
        t = (s_c2 - 2.0 * a * s_c + n_b * a * a
             + 2.0 * e * (s_cy - a * s_y) + e * e * s_y)
        t = jnp.where(n_b < 0.5, 0.0, t)
        out_ref[...] = jnp.sum(t, axis=1, keepdims=True) / float(N)


def _bin_call(conft, labels3d, edges):
    return pl.pallas_call(
        _bin_body,
        grid=(NDSTEP,),
        in_specs=[
            pl.BlockSpec((C, ND), lambda i: (0, i)),
            pl.BlockSpec((1, 1, ND), lambda i: (i, 0, 0)),
            pl.BlockSpec((C, 16), lambda i: (0, 0)),
        ],
        out_specs=pl.BlockSpec((C, 1), lambda i: (0, 0)),
        out_shape=jax.ShapeDtypeStruct((C, 1), jnp.float32),
        scratch_shapes=[
            pltpu.VMEM((C, 16), jnp.float32),
            pltpu.VMEM((C, 16), jnp.float32),
            pltpu.VMEM((C, 16), jnp.float32),
            pltpu.VMEM((C, 16), jnp.float32),
            pltpu.VMEM((C, 16), jnp.float32),
        ],
    )(conft, labels3d, edges)


def kernel(logits_target, labels_target):
    conft, idxt, minv, maxv = _softmax_call(logits_target)
    hist = _hist_call(idxt)
    edges = _edges_call(hist, minv, maxv)
    labels3d = labels_target.reshape(NDSTEP, 1, ND)
    out = _bin_call(conft, labels3d, edges)
    return out.reshape(C)


# speedup vs baseline: 10.7578x; 1.0824x over previous
import jax
import jax.numpy as jnp
from jax import lax
from jax.experimental import pallas as pl
from jax.experimental.pallas import tpu as pltpu
from jax.experimental.pallas import tpu_sc as plsc

N = 65536
C = 100
NBINS = 15
SHIFT = 16
NBUCKETS = 16384

NBA = 1024


def _softmax_body(logits_ref, conft_ref, idxt_ref, minv_ref, maxv_ref):
    i = pl.program_id(0)
    x = logits_ref[...]
    m = jnp.max(x, axis=1, keepdims=True)
    e = jnp.exp(x - m)
    s = jnp.sum(e, axis=1, keepdims=True)
    conf = e / s
    confp = jnp.pad(conf, ((0, 0), (0, 128 - C)))
    conft = confp.T[0:C, :]
    conft_ref[...] = conft
    bits = lax.bitcast_convert_type(conft, jnp.int32)
    bucket = lax.shift_right_logical(bits, SHIFT)
    rows = lax.broadcasted_iota(jnp.int32, (C, 1), 0)
    local = jnp.where(rows >= C // 2, rows - C // 2, rows)
    idxt_ref[...] = bucket + local * NBUCKETS
    mn = jnp.min(conft, axis=1, keepdims=True)
    mx = jnp.max(conft, axis=1, keepdims=True)

    @pl.when(i == 0)
    def _():
        minv_ref[...] = mn
        maxv_ref[...] = mx

    @pl.when(i > 0)
    def _():
        minv_ref[...] = jnp.minimum(minv_ref[...], mn)
        maxv_ref[...] = jnp.maximum(maxv_ref[...], mx)


def _softmax_call(logits):
    return pl.pallas_call(
        _softmax_body,
        grid=(N // NBA,),
        in_specs=[pl.BlockSpec((NBA, C), lambda i: (i, 0))],
        out_specs=[
            pl.BlockSpec((C, NBA), lambda i: (0, i)),
            pl.BlockSpec((C, NBA), lambda i: (0, i)),
            pl.BlockSpec((C, 1), lambda i: (0, 0)),
            pl.BlockSpec((C, 1), lambda i: (0, 0)),
        ],
        out_shape=[
            jax.ShapeDtypeStruct((C, N), jnp.float32),
            jax.ShapeDtypeStruct((C, N), jnp.int32),
            jax.ShapeDtypeStruct((C, 1), jnp.float32),
            jax.ShapeDtypeStruct((C, 1), jnp.float32),
        ],
    )(logits)


NCORES = 2
NSUB = 16
CLS_PER_CORE = C // NCORES
CH = 8192
NCHUNK = N // CH
ROWS = CH // 128


def _hist_body(idx2_hbm, ones_hbm, zeros_hbm, out_hbm,
               idx_v, stage_v, ones_v, sem, hist_sh):
    cid = lax.axis_index("c")
    sid = lax.axis_index("s")
    cnt = jnp.where(sid < 2, 4, 3)
    start_local = 3 * sid + jnp.minimum(sid, 2)
    core_base = cid * CLS_PER_CORE

    pltpu.sync_copy(ones_hbm, ones_v)

    def do_class(z):
        local = start_local + z
        cls = core_base + local
        hbase = local * NBUCKETS
        pltpu.sync_copy(zeros_hbm, hist_sh.at[pl.ds(hbase, NBUCKETS)])
        for ch in range(NCHUNK):
            pltpu.sync_copy(idx2_hbm.at[cls, pl.ds(ch * CH, CH)], idx_v)
            pltpu.sync_copy(ones_v, hist_sh.at[plsc.Indices(idx_v)],
                            add=True)
        pltpu.sync_copy(hist_sh.at[pl.ds(hbase, NBUCKETS)], stage_v)
        pltpu.sync_copy(stage_v, out_hbm.at[cls])

    def loop_body(z, _):
        @pl.when(z < cnt)
        def _():
            do_class(z)
        return 0

    lax.fori_loop(0, 4, loop_body, 0)


def _hist_call(idxt):
    mesh = plsc.VectorSubcoreMesh(core_axis_name="c", subcore_axis_name="s")
    ones = jnp.ones((CH,), jnp.float32)
    zeros = jnp.zeros((NBUCKETS,), jnp.float32)
    fn = pl.kernel(
        _hist_body,
        out_type=jax.ShapeDtypeStruct((C, NBUCKETS), jnp.float32),
        mesh=mesh,
        scratch_types=[
            pltpu.VMEM((CH,), jnp.int32),
            pltpu.VMEM((NBUCKETS,), jnp.float32),
            pltpu.VMEM((CH,), jnp.float32),
            pltpu.SemaphoreType.DMA,
            pltpu.VMEM_SHARED((CLS_PER_CORE * NBUCKETS,), jnp.float32),
        ],
    )
    return fn(idxt, ones, zeros)


BCH = 512
NBSTEP = NBUCKETS // BCH
BIG = 3.0e38


def _edges_body(hist_ref, minv_ref, maxv_ref, edges_ref,
                carry_ref, bacc_ref, cumlo_ref, cumhi_ref):
    j = pl.program_id(0)

    @pl.when(j == 0)
    def _():
        carry_ref[...] = jnp.zeros((C, 1), jnp.float32)
        bacc_ref[...] = jnp.zeros((C, 16), jnp.float32)
        cumlo_ref[...] = jnp.full((C, 16), -1.0, jnp.float32)
        cumhi_ref[...] = jnp.full((C, 16), BIG, jnp.float32)

    h = hist_ref[...]
    rr = lax.broadcasted_iota(jnp.int32, (BCH, BCH), 0)
    cc = lax.broadcasted_iota(jnp.int32, (BCH, BCH), 1)
    tri = (rr <= cc).astype(jnp.float32)
    cum_in = jax.lax.dot(h, tri, preferred_element_type=jnp.float32)
    cum = cum_in + carry_ref[...]
    carry_ref[...] = cum[:, BCH - 1:BCH]

    zero1 = jnp.zeros((C, 1), jnp.float32)
    neg1 = jnp.full((C, 1), -1.0, jnp.float32)
    big1 = jnp.full((C, 1), BIG, jnp.float32)
    nlt_cols, mlo_cols, mhi_cols = [zero1], [neg1], [big1]
    for k in range(1, 15):
        t = k * (float(N) / NBINS) + 0.5
        lt = cum < t
        nlt_cols.append(jnp.sum(jnp.where(lt, 1.0, 0.0), axis=1,
                                keepdims=True))
        mlo_cols.append(jnp.max(jnp.where(lt, cum, -1.0), axis=1,
                                keepdims=True))
        mhi_cols.append(jnp.min(jnp.where(lt, BIG, cum), axis=1,
                                keepdims=True))
    nlt_cols.append(zero1)
    mlo_cols.append(neg1)
    mhi_cols.append(big1)
    bacc_ref[...] = bacc_ref[...] + jnp.concatenate(nlt_cols, axis=1)
    cumlo_ref[...] = jnp.maximum(cumlo_ref[...],
                                 jnp.concatenate(mlo_cols, axis=1))
    cumhi_ref[...] = jnp.minimum(cumhi_ref[...],
                                 jnp.concatenate(mhi_cols, axis=1))

    @pl.when(j == NBSTEP - 1)
    def _():
        b_i = bacc_ref[...].astype(jnp.int32)
        vlo = lax.bitcast_convert_type(b_i << SHIFT, jnp.float32)
        vhi = lax.bitcast_convert_type((b_i + 1) << SHIFT, jnp.float32)
        lo = jnp.maximum(cumlo_ref[...], 0.0)
        hi = cumhi_ref[...]
        kk = lax.broadcasted_iota(jnp.int32, (C, 16), 1).astype(jnp.float32)
        pos = kk * (float(N) / NBINS) + 0.5
        frac = (pos - lo) / jnp.maximum(hi - lo, 1.0)
        edge = vlo + frac * (vhi - vlo)
        ki = lax.broadcasted_iota(jnp.int32, (C, 16), 1)
        mn = jnp.broadcast_to(minv_ref[...], (C, 16))
        mx = jnp.broadcast_to(maxv_ref[...], (C, 16))
        edges_ref[...] = jnp.where(ki == 0, mn,
                                   jnp.where(ki == 15, mx, edge))


def _edges_call(hist, minv, maxv):
    return pl.pallas_call(
        _edges_body,
        grid=(NBSTEP,),
        in_specs=[
            pl.BlockSpec((C, BCH), lambda j: (0, j)),
            pl.BlockSpec((C, 1), lambda j: (0, 0)),
            pl.BlockSpec((C, 1), lambda j: (0, 0)),
        ],
        out_specs=pl.BlockSpec((C, 16), lambda j: (0, 0)),
        out_shape=jax.ShapeDtypeStruct((C, 16), jnp.float32),
        scratch_shapes=[
            pltpu.VMEM((C, 1), jnp.float32),
            pltpu.VMEM((C, 16), jnp.float32),
            pltpu.VMEM((C, 16), jnp.float32),
            pltpu.VMEM((C, 16), jnp.float32),
        ],
    )(hist, minv, maxv)


ND = 2048
NDSTEP = N // ND


def _bin_body(conft_ref, labels_ref, edges_ref, out_ref,
              u1_ref, uc_ref, uc2_ref, uy_ref, ucy_ref):
    i = pl.program_id(0)

    @pl.when(i == 0)
    def _():
        z = jnp.zeros((C, 16), jnp.float32)
        u1_ref[...] = z
        uc_ref[...] = z
        uc2_ref[...] = z
        uy_ref[...] = z
        ucy_ref[...] = z

    conf = conft_ref[...]
    labels = labels_ref[0]
    clsid = lax.broadcasted_iota(jnp.int32, (C, ND), 0)
    y = (labels == clsid).astype(jnp.float32)
    c2 = conf * conf
    cy = conf * y
    edges = edges_ref[...]

    cols = []
    for k in range(16):
        m = conf > edges[:, k:k + 1]
        s1 = jnp.sum(jnp.where(m, 1.0, 0.0), axis=1, keepdims=True)
        sc = jnp.sum(jnp.where(m, conf, 0.0), axis=1, keepdims=True)
        sc2 = jnp.sum(jnp.where(m, c2, 0.0), axis=1, keepdims=True)
        sy = jnp.sum(jnp.where(m, y, 0.0), axis=1, keepdims=True)
        scy = jnp.sum(jnp.where(m, cy, 0.0), axis=1, keepdims=True)
        cols.append((s1, sc, sc2, sy, scy))
    u1_ref[...] = u1_ref[...] + jnp.concatenate([c[0] for c in cols], axis=1)
    uc_ref[...] = uc_ref[...] + jnp.concatenate([c[1] for c in cols], axis=1)
    uc2_ref[...] = uc2_ref[...] + jnp.concatenate([c[2] for c in cols],
                                                  axis=1)
    uy_ref[...] = uy_ref[...] + jnp.concatenate([c[3] for c in cols], axis=1)
    ucy_ref[...] = ucy_ref[...] + jnp.concatenate([c[4] for c in cols],
                                                  axis=1)

    @pl.when(i == NDSTEP - 1)
    def _():
        u1 = u1_ref[...]
        uc = uc_ref[...]
        uc2 = uc2_ref[...]
        uy = uy_ref[...]
        ucy = ucy_ref[...]
        n_b = u1[:, 0:15] - u1[:, 1:16]
        s_c = uc[:, 0:15] - uc[:, 1:16]
        s_c2 = uc2[:, 0:15] - uc2[:, 1:16]
        s_y = uy[:, 0:15] - uy[:, 1:16]
        s_cy = ucy[:, 0:15] - ucy[:, 1:16]
        e = 1.0 / (n_b - 1.0)
        a = s_y * e
        t = (s_c2 - 2.0 * a * s_c + n_b * a * a
             + 2.0 * e * (s_cy - a * s_y) + e * e * s_y)
        t = jnp.where(n_b < 0.5, 0.0, t)
        out_ref[...] = jnp.sum(t, axis=1, keepdims=True) / float(N)


def _bin_call(conft, labels3d, edges):
    return pl.pallas_call(
        _bin_body,
        grid=(NDSTEP,),
        in_specs=[
            pl.BlockSpec((C, ND), lambda i: (0, i)),
            pl.BlockSpec((1, 1, ND), lambda i: (i, 0, 0)),
            pl.BlockSpec((C, 16), lambda i: (0, 0)),
        ],
        out_specs=pl.BlockSpec((C, 1), lambda i: (0, 0)),
        out_shape=jax.ShapeDtypeStruct((C, 1), jnp.float32),
        scratch_shapes=[
            pltpu.VMEM((C, 16), jnp.float32),
            pltpu.VMEM((C, 16), jnp.float32),
            pltpu.VMEM((C, 16), jnp.float32),
            pltpu.VMEM((C, 16), jnp.float32),
            pltpu.VMEM((C, 16), jnp.float32),
        ],
    )(conft, labels3d, edges)


def kernel(logits_target, labels_target):
    conft, idxt, minv, maxv = _softmax_call(logits_target)
    hist = _hist_call(idxt)
    edges = _edges_call(hist, minv, maxv)
    labels3d = labels_target.reshape(NDSTEP, 1, ND)
    out = _bin_call(conft, labels3d, edges)
    return out.reshape(C)
